# baseline (speedup 1.0000x reference)
import functools

import jax
import jax.numpy as jnp
from jax import lax
from jax.experimental import pallas as pl
from jax.experimental.pallas import tpu as pltpu
from jax.experimental.pallas import tpu_sc as plsc

B, N, S, D1, D2, K = 8, 4096, 1024, 128, 256, 5
C0, C1 = 256, 128
BN = 4096
Q = B * N
KP = 8
INF = 3e38
BIG = 1e37
EPS = 1.1920928955078125e-07

NC, NS = 2, 16
NW = NC * NS
QPW = Q // NW
CQ = 16
NCHUNK = QPW // CQ


def _knn_idx_body(plens_ref, elens_ref, xyz1_ref, xyz2t_ref, gidx_ref, wn_ref):
    b = pl.program_id(0)

    a = xyz1_ref[0]
    bt = xyz2t_ref[0]
    an = jnp.sum(a * a, axis=1, keepdims=True)
    bn_sq = jnp.sum(bt * bt, axis=0, keepdims=True)
    col_s = jax.lax.broadcasted_iota(jnp.int32, (1, S), 1)
    elen = elens_ref[b]
    bn_m = jnp.where(col_s < elen, bn_sq, BIG)
    ab2 = jax.lax.dot_general(a * (-2.0), bt, (((1,), (0,)), ((), ())),
                              precision=jax.lax.Precision.HIGHEST,
                              preferred_element_type=jnp.float32)
    d0 = jnp.maximum(ab2 + an + bn_m, 0.0)

    col = jax.lax.broadcasted_iota(jnp.int32, (BN, S), 1)
    lane8 = jax.lax.broadcasted_iota(jnp.int32, (BN, KP), 1)
    d = d0
    acc_idx = jnp.zeros((BN, KP), jnp.int32)
    acc_w = jnp.zeros((BN, KP), jnp.float32)
    for k in range(K):
        m = jnp.min(d, axis=1, keepdims=True)
        pick_col = jnp.min(jnp.where(d == m, col, S), axis=1,
                           keepdims=True)
        wk = 1.0 / (m + EPS)
        acc_idx = acc_idx + jnp.where(lane8 == k,
                                      jnp.broadcast_to(pick_col, (BN, KP)), 0)
        acc_w = acc_w + jnp.where(lane8 == k,
                                  jnp.broadcast_to(wk, (BN, KP)), 0.0)
        d = jnp.where(col == pick_col, INF, d)
    wsum = jnp.sum(acc_w, axis=1, keepdims=True)
    wn_ref[0] = acc_w / wsum
    gidx_ref[0] = acc_idx + b * S


def _sc_gather_body(idx_hbm, w_hbm, table_hbm, out_hbm, idx_v, w_v, rows0_v,
                    rows1_v, out_v, sem0, sem1):
    wid = lax.axis_index("s") * NC + lax.axis_index("c")
    base_q = wid * QPW
    rows = (rows0_v, rows1_v)
    sems = (sem0, sem1)

    pltpu.sync_copy(idx_hbm.at[pl.ds(wid * NCHUNK, NCHUNK)], idx_v)
    pltpu.sync_copy(w_hbm.at[pl.ds(wid * NCHUNK, NCHUNK)], w_v)
    pltpu.async_copy(table_hbm.at[idx_v.at[0]], rows0_v, sem0)

    def pair(i, carry):
        for p in range(2):
            c = 2 * i + p
            q0 = base_q + c * CQ

            @pl.when(c + 1 < NCHUNK)
            def _():
                pltpu.async_copy(table_hbm.at[idx_v.at[c + 1]],
                                 rows[1 - p], sems[1 - p])

            pltpu.make_async_copy(table_hbm.at[idx_v.at[c]], rows[p],
                                  sems[p]).wait()

            def per_q2(q2, carry2, *, p=p, c=c):
                wv = w_v[c, pl.ds(q2 * 16, 16)]
                for half in range(2):
                    q = 2 * q2 + half
                    r0 = q * KP
                    for j in range(D2 // 16):
                        sl = pl.ds(j * 16, 16)
                        acc = wv[half * KP] * rows[p][r0, sl]
                        for k in range(1, KP):
                            acc = acc + wv[half * KP + k] * rows[p][r0 + k, sl]
                        out_v[q, sl] = acc
                return carry2

            lax.fori_loop(0, CQ // 2, per_q2, 0)
            pltpu.sync_copy(out_v, out_hbm.at[pl.ds(q0, CQ)])
        return carry

    lax.fori_loop(0, NCHUNK // 2, pair, 0)


_sc_gather = functools.partial(
    pl.kernel,
    out_type=jax.ShapeDtypeStruct((Q, D2), jnp.float32),
    mesh=plsc.VectorSubcoreMesh(core_axis_name="c", subcore_axis_name="s"),
    scratch_types=[
        pltpu.VMEM((NCHUNK, CQ * KP), jnp.int32),
        pltpu.VMEM((NCHUNK, CQ * KP), jnp.float32),
        pltpu.VMEM((CQ * KP, D2), jnp.float32),
        pltpu.VMEM((CQ * KP, D2), jnp.float32),
        pltpu.VMEM((CQ, D2), jnp.float32),
        pltpu.SemaphoreType.DMA,
        pltpu.SemaphoreType.DMA,
    ],
)(_sc_gather_body)


def _h0_body(plens_ref, p1_ref, interp_ref, w0t_ref, h0_ref, stats_ref):
    b = pl.program_id(0)
    nb = pl.program_id(1)
    w0t = w0t_ref[...]
    h0 = (jax.lax.dot_general(p1_ref[0], w0t[:D1], (((1,), (0,)), ((), ())),
                              preferred_element_type=jnp.float32)
          + jax.lax.dot_general(interp_ref[0], w0t[D1:],
                                (((1,), (0,)), ((), ())),
                                preferred_element_type=jnp.float32))
    h0_ref[0] = h0

    row = jax.lax.broadcasted_iota(jnp.int32, (BN, 1), 0) + nb * BN
    m_row = (row < plens_ref[b]).astype(jnp.float32)
    s1 = jnp.sum(h0 * m_row, axis=0, keepdims=True)
    s2 = jnp.sum(h0 * h0 * m_row, axis=0, keepdims=True)
    riota = jax.lax.broadcasted_iota(jnp.int32, (8, C0), 0)
    contrib = (jnp.where(riota == 0, jnp.broadcast_to(s1, (8, C0)), 0.0)
               + jnp.where(riota == 1, jnp.broadcast_to(s2, (8, C0)), 0.0))

    @pl.when(jnp.logical_and(b == 0, nb == 0))
    def _():
        stats_ref[...] = jnp.zeros((8, C0), jnp.float32)

    stats_ref[...] += contrib


def _n_valid(plens_ref):
    nv = jnp.int32(0)
    for i in range(B):
        nv = nv + plens_ref[i]
    return jnp.maximum(nv.astype(jnp.float32), 1.0)


def _bn_mlp_body(plens_ref, h0_ref, stats_ref, g_ref, bias_ref, w1t_ref,
                 h1_ref, stats2_ref, *, cin, cout):
    b = pl.program_id(0)
    nb = pl.program_id(1)
    nv = _n_valid(plens_ref)
    stats = stats_ref[...]
    mean = stats[0:1, :] / nv
    var = stats[1:2, :] / nv - mean * mean
    scale = g_ref[...] * jax.lax.rsqrt(var + 1e-5)
    shift = bias_ref[...] - mean * scale
    xn = jnp.maximum(h0_ref[0] * scale + shift, 0.0)
    h1 = jax.lax.dot_general(xn, w1t_ref[...], (((1,), (0,)), ((), ())),
                             preferred_element_type=jnp.float32)
    h1_ref[0] = h1

    row = jax.lax.broadcasted_iota(jnp.int32, (BN, 1), 0) + nb * BN
    m_row = (row < plens_ref[b]).astype(jnp.float32)
    s1 = jnp.sum(h1 * m_row, axis=0, keepdims=True)
    s2 = jnp.sum(h1 * h1 * m_row, axis=0, keepdims=True)
    riota = jax.lax.broadcasted_iota(jnp.int32, (8, cout), 0)
    contrib = (jnp.where(riota == 0, jnp.broadcast_to(s1, (8, cout)), 0.0)
               + jnp.where(riota == 1, jnp.broadcast_to(s2, (8, cout)), 0.0))

    @pl.when(jnp.logical_and(b == 0, nb == 0))
    def _():
        stats2_ref[...] = jnp.zeros((8, cout), jnp.float32)

    stats2_ref[...] += contrib


def _bn_final_body(plens_ref, h1_ref, stats_ref, g_ref, bias_ref, out_ref, *,
                   cin):
    nv = _n_valid(plens_ref)
    stats = stats_ref[...]
    mean = stats[0:1, :] / nv
    var = stats[1:2, :] / nv - mean * mean
    scale = g_ref[...] * jax.lax.rsqrt(var + 1e-5)
    shift = bias_ref[...] - mean * scale
    out_ref[0] = jnp.maximum(h1_ref[0] * scale + shift, 0.0)


def kernel(xyz1, xyz2, points1, points2, point_lens, embedding_lens,
           point_mask, W0, g0, b0, W1, g1, b1):
    del point_mask
    xyz2t = xyz2.transpose(0, 2, 1)
    w0t = W0.T
    w1t = W1.T
    g0r, b0r = g0.reshape(1, C0), b0.reshape(1, C0)
    g1r, b1r = g1.reshape(1, C1), b1.reshape(1, C1)
    p2_flat = points2.reshape(B * S, D2)

    grid = (B, N // BN)
    smem = pl.BlockSpec(memory_space=pltpu.SMEM)

    gidx, wn = pl.pallas_call(
        _knn_idx_body,
        grid=grid,
        in_specs=[
            smem, smem,
            pl.BlockSpec((1, BN, 3), lambda b, n: (b, n, 0)),
            pl.BlockSpec((1, 3, S), lambda b, n: (b, 0, 0)),
        ],
        out_specs=[
            pl.BlockSpec((1, BN, KP), lambda b, n: (b, n, 0)),
            pl.BlockSpec((1, BN, KP), lambda b, n: (b, n, 0)),
        ],
        out_shape=[
            jax.ShapeDtypeStruct((B, N, KP), jnp.int32),
            jax.ShapeDtypeStruct((B, N, KP), jnp.float32),
        ],
    )(point_lens, embedding_lens, xyz1, xyz2t)

    interp = _sc_gather(gidx.reshape(NW * NCHUNK, CQ * KP),
                        wn.reshape(NW * NCHUNK, CQ * KP), p2_flat)
    interp = interp.reshape(B, N, D2)

    h0, stats0 = pl.pallas_call(
        _h0_body,
        grid=grid,
        in_specs=[
            smem,
            pl.BlockSpec((1, BN, D1), lambda b, n: (b, n, 0)),
            pl.BlockSpec((1, BN, D2), lambda b, n: (b, n, 0)),
            pl.BlockSpec((D1 + D2, C0), lambda b, n: (0, 0)),
        ],
        out_specs=[
            pl.BlockSpec((1, BN, C0), lambda b, n: (b, n, 0)),
            pl.BlockSpec((8, C0), lambda b, n: (0, 0)),
        ],
        out_shape=[
            jax.ShapeDtypeStruct((B, N, C0), jnp.float32),
            jax.ShapeDtypeStruct((8, C0), jnp.float32),
        ],
    )(point_lens, points1, interp, w0t)

    h1, stats1 = pl.pallas_call(
        functools.partial(_bn_mlp_body, cin=C0, cout=C1),
        grid=grid,
        in_specs=[
            smem,
            pl.BlockSpec((1, BN, C0), lambda b, n: (b, n, 0)),
            pl.BlockSpec((8, C0), lambda b, n: (0, 0)),
            pl.BlockSpec((1, C0), lambda b, n: (0, 0)),
            pl.BlockSpec((1, C0), lambda b, n: (0, 0)),
            pl.BlockSpec((C0, C1), lambda b, n: (0, 0)),
        ],
        out_specs=[
            pl.BlockSpec((1, BN, C1), lambda b, n: (b, n, 0)),
            pl.BlockSpec((8, C1), lambda b, n: (0, 0)),
        ],
        out_shape=[
            jax.ShapeDtypeStruct((B, N, C1), jnp.float32),
            jax.ShapeDtypeStruct((8, C1), jnp.float32),
        ],
    )(point_lens, h0, stats0, g0r, b0r, w1t)

    out = pl.pallas_call(
        functools.partial(_bn_final_body, cin=C1),
        grid=grid,
        in_specs=[
            smem,
            pl.BlockSpec((1, BN, C1), lambda b, n: (b, n, 0)),
            pl.BlockSpec((8, C1), lambda b, n: (0, 0)),
            pl.BlockSpec((1, C1), lambda b, n: (0, 0)),
            pl.BlockSpec((1, C1), lambda b, n: (0, 0)),
        ],
        out_specs=pl.BlockSpec((1, BN, C1), lambda b, n: (b, n, 0)),
        out_shape=jax.ShapeDtypeStruct((B, N, C1), jnp.float32),
    )(point_lens, h1, stats1, g1r, b1r)

    return out


# speedup vs baseline: 1.0016x; 1.0016x over previous
import functools

import jax
import jax.numpy as jnp
from jax import lax
from jax.experimental import pallas as pl
from jax.experimental.pallas import tpu as pltpu
from jax.experimental.pallas import tpu_sc as plsc

B, N, S, D1, D2, K = 8, 4096, 1024, 128, 256, 5
C0, C1 = 256, 128
BN = 4096
Q = B * N
KP = 8
INF = 3e38
BIG = 1e37
EPS = 1.1920928955078125e-07

NC, NS = 2, 16
NW = NC * NS
QPW = Q // NW
CQ = 16
NCHUNK = QPW // CQ


def _knn_idx_body(plens_ref, elens_ref, xyz1_ref, xyz2t_ref, gidx_ref, wn_ref):
    b = pl.program_id(0)

    a = xyz1_ref[0]
    bt = xyz2t_ref[0]
    an = jnp.sum(a * a, axis=1, keepdims=True)
    bn_sq = jnp.sum(bt * bt, axis=0, keepdims=True)
    col_s = jax.lax.broadcasted_iota(jnp.int32, (1, S), 1)
    elen = elens_ref[b]
    bn_m = jnp.where(col_s < elen, bn_sq, BIG)
    ab2 = jax.lax.dot_general(a * (-2.0), bt, (((1,), (0,)), ((), ())),
                              precision=jax.lax.Precision.HIGHEST,
                              preferred_element_type=jnp.float32)
    d0 = jnp.maximum(ab2 + an + bn_m, 0.0)

    col = jax.lax.broadcasted_iota(jnp.int32, (BN, S), 1)
    lane8 = jax.lax.broadcasted_iota(jnp.int32, (BN, KP), 1)
    d = d0
    acc_idx = jnp.zeros((BN, KP), jnp.int32)
    acc_w = jnp.zeros((BN, KP), jnp.float32)
    for k in range(K):
        m = jnp.min(d, axis=1, keepdims=True)
        pick_col = jnp.min(jnp.where(d == m, col, S), axis=1,
                           keepdims=True)
        wk = 1.0 / (m + EPS)
        acc_idx = acc_idx + jnp.where(lane8 == k,
                                      jnp.broadcast_to(pick_col, (BN, KP)), 0)
        acc_w = acc_w + jnp.where(lane8 == k,
                                  jnp.broadcast_to(wk, (BN, KP)), 0.0)
        d = jnp.where(col == pick_col, INF, d)
    wsum = jnp.sum(acc_w, axis=1, keepdims=True)
    wn_ref[0] = acc_w / wsum
    gidx_ref[0] = acc_idx + b * S


def _sc_gather_body(idx_hbm, w_hbm, table_hbm, out_hbm, idx_v, w_v, rows0_v,
                    rows1_v, out_v, sem0, sem1):
    wid = lax.axis_index("s") * NC + lax.axis_index("c")
    base_q = wid * QPW
    rows = (rows0_v, rows1_v)
    sems = (sem0, sem1)

    pltpu.sync_copy(idx_hbm.at[pl.ds(wid * NCHUNK, NCHUNK)], idx_v)
    pltpu.sync_copy(w_hbm.at[pl.ds(wid * NCHUNK, NCHUNK)], w_v)
    pltpu.async_copy(table_hbm.at[idx_v.at[0]], rows0_v, sem0)

    def pair(i, carry):
        for p in range(2):
            c = 2 * i + p
            q0 = base_q + c * CQ

            @pl.when(c + 1 < NCHUNK)
            def _():
                pltpu.async_copy(table_hbm.at[idx_v.at[c + 1]],
                                 rows[1 - p], sems[1 - p])

            pltpu.make_async_copy(table_hbm.at[idx_v.at[c]], rows[p],
                                  sems[p]).wait()

            def per_q2(q2, carry2, *, p=p, c=c):
                wv = w_v[c, pl.ds(q2 * 16, 16)]
                for half in range(2):
                    q = 2 * q2 + half
                    r0 = q * KP
                    for j in range(D2 // 16):
                        sl = pl.ds(j * 16, 16)
                        h = half * KP
                        t0 = (wv[h] * rows[p][r0, sl]
                              + wv[h + 1] * rows[p][r0 + 1, sl])
                        t1 = (wv[h + 2] * rows[p][r0 + 2, sl]
                              + wv[h + 3] * rows[p][r0 + 3, sl])
                        out_v[q, sl] = (t0 + t1) + wv[h + 4] * rows[p][r0 + 4, sl]
                return carry2

            lax.fori_loop(0, CQ // 2, per_q2, 0)
            pltpu.sync_copy(out_v, out_hbm.at[pl.ds(q0, CQ)])
        return carry

    lax.fori_loop(0, NCHUNK // 2, pair, 0)


_sc_gather = functools.partial(
    pl.kernel,
    out_type=jax.ShapeDtypeStruct((Q, D2), jnp.float32),
    mesh=plsc.VectorSubcoreMesh(core_axis_name="c", subcore_axis_name="s"),
    scratch_types=[
        pltpu.VMEM((NCHUNK, CQ * KP), jnp.int32),
        pltpu.VMEM((NCHUNK, CQ * KP), jnp.float32),
        pltpu.VMEM((CQ * KP, D2), jnp.float32),
        pltpu.VMEM((CQ * KP, D2), jnp.float32),
        pltpu.VMEM((CQ, D2), jnp.float32),
        pltpu.SemaphoreType.DMA,
        pltpu.SemaphoreType.DMA,
    ],
)(_sc_gather_body)


def _h0_body(plens_ref, p1_ref, interp_ref, w0t_ref, h0_ref, stats_ref):
    b = pl.program_id(0)
    nb = pl.program_id(1)
    w0t = w0t_ref[...]
    h0 = (jax.lax.dot_general(p1_ref[0], w0t[:D1], (((1,), (0,)), ((), ())),
                              preferred_element_type=jnp.float32)
          + jax.lax.dot_general(interp_ref[0], w0t[D1:],
                                (((1,), (0,)), ((), ())),
                                preferred_element_type=jnp.float32))
    h0_ref[0] = h0

    row = jax.lax.broadcasted_iota(jnp.int32, (BN, 1), 0) + nb * BN
    m_row = (row < plens_ref[b]).astype(jnp.float32)
    s1 = jnp.sum(h0 * m_row, axis=0, keepdims=True)
    s2 = jnp.sum(h0 * h0 * m_row, axis=0, keepdims=True)
    riota = jax.lax.broadcasted_iota(jnp.int32, (8, C0), 0)
    contrib = (jnp.where(riota == 0, jnp.broadcast_to(s1, (8, C0)), 0.0)
               + jnp.where(riota == 1, jnp.broadcast_to(s2, (8, C0)), 0.0))

    @pl.when(jnp.logical_and(b == 0, nb == 0))
    def _():
        stats_ref[...] = jnp.zeros((8, C0), jnp.float32)

    stats_ref[...] += contrib


def _n_valid(plens_ref):
    nv = jnp.int32(0)
    for i in range(B):
        nv = nv + plens_ref[i]
    return jnp.maximum(nv.astype(jnp.float32), 1.0)


def _bn_mlp_body(plens_ref, h0_ref, stats_ref, g_ref, bias_ref, w1t_ref,
                 h1_ref, stats2_ref, *, cin, cout):
    b = pl.program_id(0)
    nb = pl.program_id(1)
    nv = _n_valid(plens_ref)
    stats = stats_ref[...]
    mean = stats[0:1, :] / nv
    var = stats[1:2, :] / nv - mean * mean
    scale = g_ref[...] * jax.lax.rsqrt(var + 1e-5)
    shift = bias_ref[...] - mean * scale
    xn = jnp.maximum(h0_ref[0] * scale + shift, 0.0)
    h1 = jax.lax.dot_general(xn, w1t_ref[...], (((1,), (0,)), ((), ())),
                             preferred_element_type=jnp.float32)
    h1_ref[0] = h1

    row = jax.lax.broadcasted_iota(jnp.int32, (BN, 1), 0) + nb * BN
    m_row = (row < plens_ref[b]).astype(jnp.float32)
    s1 = jnp.sum(h1 * m_row, axis=0, keepdims=True)
    s2 = jnp.sum(h1 * h1 * m_row, axis=0, keepdims=True)
    riota = jax.lax.broadcasted_iota(jnp.int32, (8, cout), 0)
    contrib = (jnp.where(riota == 0, jnp.broadcast_to(s1, (8, cout)), 0.0)
               + jnp.where(riota == 1, jnp.broadcast_to(s2, (8, cout)), 0.0))

    @pl.when(jnp.logical_and(b == 0, nb == 0))
    def _():
        stats2_ref[...] = jnp.zeros((8, cout), jnp.float32)

    stats2_ref[...] += contrib


def _bn_final_body(plens_ref, h1_ref, stats_ref, g_ref, bias_ref, out_ref, *,
                   cin):
    nv = _n_valid(plens_ref)
    stats = stats_ref[...]
    mean = stats[0:1, :] / nv
    var = stats[1:2, :] / nv - mean * mean
    scale = g_ref[...] * jax.lax.rsqrt(var + 1e-5)
    shift = bias_ref[...] - mean * scale
    out_ref[0] = jnp.maximum(h1_ref[0] * scale + shift, 0.0)


def kernel(xyz1, xyz2, points1, points2, point_lens, embedding_lens,
           point_mask, W0, g0, b0, W1, g1, b1):
    del point_mask
    xyz2t = xyz2.transpose(0, 2, 1)
    w0t = W0.T
    w1t = W1.T
    g0r, b0r = g0.reshape(1, C0), b0.reshape(1, C0)
    g1r, b1r = g1.reshape(1, C1), b1.reshape(1, C1)
    p2_flat = points2.reshape(B * S, D2)

    grid = (B, N // BN)
    smem = pl.BlockSpec(memory_space=pltpu.SMEM)

    gidx, wn = pl.pallas_call(
        _knn_idx_body,
        grid=grid,
        in_specs=[
            smem, smem,
            pl.BlockSpec((1, BN, 3), lambda b, n: (b, n, 0)),
            pl.BlockSpec((1, 3, S), lambda b, n: (b, 0, 0)),
        ],
        out_specs=[
            pl.BlockSpec((1, BN, KP), lambda b, n: (b, n, 0)),
            pl.BlockSpec((1, BN, KP), lambda b, n: (b, n, 0)),
        ],
        out_shape=[
            jax.ShapeDtypeStruct((B, N, KP), jnp.int32),
            jax.ShapeDtypeStruct((B, N, KP), jnp.float32),
        ],
    )(point_lens, embedding_lens, xyz1, xyz2t)

    interp = _sc_gather(gidx.reshape(NW * NCHUNK, CQ * KP),
                        wn.reshape(NW * NCHUNK, CQ * KP), p2_flat)
    interp = interp.reshape(B, N, D2)

    h0, stats0 = pl.pallas_call(
        _h0_body,
        grid=grid,
        in_specs=[
            smem,
            pl.BlockSpec((1, BN, D1), lambda b, n: (b, n, 0)),
            pl.BlockSpec((1, BN, D2), lambda b, n: (b, n, 0)),
            pl.BlockSpec((D1 + D2, C0), lambda b, n: (0, 0)),
        ],
        out_specs=[
            pl.BlockSpec((1, BN, C0), lambda b, n: (b, n, 0)),
            pl.BlockSpec((8, C0), lambda b, n: (0, 0)),
        ],
        out_shape=[
            jax.ShapeDtypeStruct((B, N, C0), jnp.float32),
            jax.ShapeDtypeStruct((8, C0), jnp.float32),
        ],
    )(point_lens, points1, interp, w0t)

    h1, stats1 = pl.pallas_call(
        functools.partial(_bn_mlp_body, cin=C0, cout=C1),
        grid=grid,
        in_specs=[
            smem,
            pl.BlockSpec((1, BN, C0), lambda b, n: (b, n, 0)),
            pl.BlockSpec((8, C0), lambda b, n: (0, 0)),
            pl.BlockSpec((1, C0), lambda b, n: (0, 0)),
            pl.BlockSpec((1, C0), lambda b, n: (0, 0)),
            pl.BlockSpec((C0, C1), lambda b, n: (0, 0)),
        ],
        out_specs=[
            pl.BlockSpec((1, BN, C1), lambda b, n: (b, n, 0)),
            pl.BlockSpec((8, C1), lambda b, n: (0, 0)),
        ],
        out_shape=[
            jax.ShapeDtypeStruct((B, N, C1), jnp.float32),
            jax.ShapeDtypeStruct((8, C1), jnp.float32),
        ],
    )(point_lens, h0, stats0, g0r, b0r, w1t)

    out = pl.pallas_call(
        functools.partial(_bn_final_body, cin=C1),
        grid=grid,
        in_specs=[
            smem,
            pl.BlockSpec((1, BN, C1), lambda b, n: (b, n, 0)),
            pl.BlockSpec((8, C1), lambda b, n: (0, 0)),
            pl.BlockSpec((1, C1), lambda b, n: (0, 0)),
            pl.BlockSpec((1, C1), lambda b, n: (0, 0)),
        ],
        out_specs=pl.BlockSpec((1, BN, C1), lambda b, n: (b, n, 0)),
        out_shape=jax.ShapeDtypeStruct((B, N, C1), jnp.float32),
    )(point_lens, h1, stats1, g1r, b1r)

    return out


# speedup vs baseline: 1.0017x; 1.0000x over previous
import functools

import jax
import jax.numpy as jnp
from jax import lax
from jax.experimental import pallas as pl
from jax.experimental.pallas import tpu as pltpu
from jax.experimental.pallas import tpu_sc as plsc

B, N, S, D1, D2, K = 8, 4096, 1024, 128, 256, 5
C0, C1 = 256, 128
BN = 4096
Q = B * N
KP = 8
INF = 3e38
BIG = 1e37
EPS = 1.1920928955078125e-07

NC, NS = 2, 16
NW = NC * NS
QPW = Q // NW
CQ = 8
NCHUNK = QPW // CQ


def _knn_idx_body(plens_ref, elens_ref, xyz1_ref, xyz2t_ref, gidx_ref, wn_ref):
    b = pl.program_id(0)

    a = xyz1_ref[0]
    bt = xyz2t_ref[0]
    an = jnp.sum(a * a, axis=1, keepdims=True)
    bn_sq = jnp.sum(bt * bt, axis=0, keepdims=True)
    col_s = jax.lax.broadcasted_iota(jnp.int32, (1, S), 1)
    elen = elens_ref[b]
    bn_m = jnp.where(col_s < elen, bn_sq, BIG)
    ab2 = jax.lax.dot_general(a * (-2.0), bt, (((1,), (0,)), ((), ())),
                              precision=jax.lax.Precision.HIGHEST,
                              preferred_element_type=jnp.float32)
    d0 = jnp.maximum(ab2 + an + bn_m, 0.0)

    col = jax.lax.broadcasted_iota(jnp.int32, (BN, S), 1)
    lane8 = jax.lax.broadcasted_iota(jnp.int32, (BN, KP), 1)
    d = d0
    acc_idx = jnp.zeros((BN, KP), jnp.int32)
    acc_w = jnp.zeros((BN, KP), jnp.float32)
    for k in range(K):
        m = jnp.min(d, axis=1, keepdims=True)
        pick_col = jnp.min(jnp.where(d == m, col, S), axis=1,
                           keepdims=True)
        wk = 1.0 / (m + EPS)
        acc_idx = acc_idx + jnp.where(lane8 == k,
                                      jnp.broadcast_to(pick_col, (BN, KP)), 0)
        acc_w = acc_w + jnp.where(lane8 == k,
                                  jnp.broadcast_to(wk, (BN, KP)), 0.0)
        d = jnp.where(col == pick_col, INF, d)
    wsum = jnp.sum(acc_w, axis=1, keepdims=True)
    wn_ref[0] = acc_w / wsum
    gidx_ref[0] = acc_idx + b * S


NRING = 4


def _sc_gather_body(idx_hbm, w_hbm, table_hbm, out_hbm, idx_v, w_v, rows0_v,
                    rows1_v, rows2_v, rows3_v, out_v, sem0, sem1, sem2, sem3):
    wid = lax.axis_index("s") * NC + lax.axis_index("c")
    base_q = wid * QPW
    rows = (rows0_v, rows1_v, rows2_v, rows3_v)
    sems = (sem0, sem1, sem2, sem3)

    pltpu.sync_copy(idx_hbm.at[pl.ds(wid * NCHUNK, NCHUNK)], idx_v)
    pltpu.sync_copy(w_hbm.at[pl.ds(wid * NCHUNK, NCHUNK)], w_v)
    for c0 in range(NRING - 1):
        pltpu.async_copy(table_hbm.at[idx_v.at[c0]], rows[c0], sems[c0])

    def ring(i, carry):
        for p in range(NRING):
            c = NRING * i + p
            q0 = base_q + c * CQ
            pf = (p + NRING - 1) % NRING

            @pl.when(c + NRING - 1 < NCHUNK)
            def _():
                pltpu.async_copy(table_hbm.at[idx_v.at[c + NRING - 1]],
                                 rows[pf], sems[pf])

            pltpu.make_async_copy(table_hbm.at[idx_v.at[c]], rows[p],
                                  sems[p]).wait()

            def per_q2(q2, carry2, *, p=p, c=c):
                wv = w_v[c, pl.ds(q2 * 16, 16)]
                for half in range(2):
                    q = 2 * q2 + half
                    r0 = q * KP
                    for j in range(D2 // 16):
                        sl = pl.ds(j * 16, 16)
                        h = half * KP
                        t0 = (wv[h] * rows[p][r0, sl]
                              + wv[h + 1] * rows[p][r0 + 1, sl])
                        t1 = (wv[h + 2] * rows[p][r0 + 2, sl]
                              + wv[h + 3] * rows[p][r0 + 3, sl])
                        out_v[q, sl] = ((t0 + t1)
                                        + wv[h + 4] * rows[p][r0 + 4, sl])
                return carry2

            lax.fori_loop(0, CQ // 2, per_q2, 0)
            pltpu.sync_copy(out_v, out_hbm.at[pl.ds(q0, CQ)])
        return carry

    lax.fori_loop(0, NCHUNK // NRING, ring, 0)


_sc_gather = functools.partial(
    pl.kernel,
    out_type=jax.ShapeDtypeStruct((Q, D2), jnp.float32),
    mesh=plsc.VectorSubcoreMesh(core_axis_name="c", subcore_axis_name="s"),
    scratch_types=[
        pltpu.VMEM((NCHUNK, CQ * KP), jnp.int32),
        pltpu.VMEM((NCHUNK, CQ * KP), jnp.float32),
        pltpu.VMEM((CQ * KP, D2), jnp.float32),
        pltpu.VMEM((CQ * KP, D2), jnp.float32),
        pltpu.VMEM((CQ * KP, D2), jnp.float32),
        pltpu.VMEM((CQ * KP, D2), jnp.float32),
        pltpu.VMEM((CQ, D2), jnp.float32),
        pltpu.SemaphoreType.DMA,
        pltpu.SemaphoreType.DMA,
        pltpu.SemaphoreType.DMA,
        pltpu.SemaphoreType.DMA,
    ],
)(_sc_gather_body)


def _h0_body(plens_ref, p1_ref, interp_ref, w0t_ref, h0_ref, stats_ref):
    b = pl.program_id(0)
    nb = pl.program_id(1)
    w0t = w0t_ref[...]
    h0 = (jax.lax.dot_general(p1_ref[0], w0t[:D1], (((1,), (0,)), ((), ())),
                              preferred_element_type=jnp.float32)
          + jax.lax.dot_general(interp_ref[0], w0t[D1:],
                                (((1,), (0,)), ((), ())),
                                preferred_element_type=jnp.float32))
    h0_ref[0] = h0

    row = jax.lax.broadcasted_iota(jnp.int32, (BN, 1), 0) + nb * BN
    m_row = (row < plens_ref[b]).astype(jnp.float32)
    s1 = jnp.sum(h0 * m_row, axis=0, keepdims=True)
    s2 = jnp.sum(h0 * h0 * m_row, axis=0, keepdims=True)
    riota = jax.lax.broadcasted_iota(jnp.int32, (8, C0), 0)
    contrib = (jnp.where(riota == 0, jnp.broadcast_to(s1, (8, C0)), 0.0)
               + jnp.where(riota == 1, jnp.broadcast_to(s2, (8, C0)), 0.0))

    @pl.when(jnp.logical_and(b == 0, nb == 0))
    def _():
        stats_ref[...] = jnp.zeros((8, C0), jnp.float32)

    stats_ref[...] += contrib


def _n_valid(plens_ref):
    nv = jnp.int32(0)
    for i in range(B):
        nv = nv + plens_ref[i]
    return jnp.maximum(nv.astype(jnp.float32), 1.0)


def _bn_mlp_body(plens_ref, h0_ref, stats_ref, g_ref, bias_ref, w1t_ref,
                 h1_ref, stats2_ref, *, cin, cout):
    b = pl.program_id(0)
    nb = pl.program_id(1)
    nv = _n_valid(plens_ref)
    stats = stats_ref[...]
    mean = stats[0:1, :] / nv
    var = stats[1:2, :] / nv - mean * mean
    scale = g_ref[...] * jax.lax.rsqrt(var + 1e-5)
    shift = bias_ref[...] - mean * scale
    xn = jnp.maximum(h0_ref[0] * scale + shift, 0.0)
    h1 = jax.lax.dot_general(xn, w1t_ref[...], (((1,), (0,)), ((), ())),
                             preferred_element_type=jnp.float32)
    h1_ref[0] = h1

    row = jax.lax.broadcasted_iota(jnp.int32, (BN, 1), 0) + nb * BN
    m_row = (row < plens_ref[b]).astype(jnp.float32)
    s1 = jnp.sum(h1 * m_row, axis=0, keepdims=True)
    s2 = jnp.sum(h1 * h1 * m_row, axis=0, keepdims=True)
    riota = jax.lax.broadcasted_iota(jnp.int32, (8, cout), 0)
    contrib = (jnp.where(riota == 0, jnp.broadcast_to(s1, (8, cout)), 0.0)
               + jnp.where(riota == 1, jnp.broadcast_to(s2, (8, cout)), 0.0))

    @pl.when(jnp.logical_and(b == 0, nb == 0))
    def _():
        stats2_ref[...] = jnp.zeros((8, cout), jnp.float32)

    stats2_ref[...] += contrib


def _bn_final_body(plens_ref, h1_ref, stats_ref, g_ref, bias_ref, out_ref, *,
                   cin):
    nv = _n_valid(plens_ref)
    stats = stats_ref[...]
    mean = stats[0:1, :] / nv
    var = stats[1:2, :] / nv - mean * mean
    scale = g_ref[...] * jax.lax.rsqrt(var + 1e-5)
    shift = bias_ref[...] - mean * scale
    out_ref[0] = jnp.maximum(h1_ref[0] * scale + shift, 0.0)


def kernel(xyz1, xyz2, points1, points2, point_lens, embedding_lens,
           point_mask, W0, g0, b0, W1, g1, b1):
    del point_mask
    xyz2t = xyz2.transpose(0, 2, 1)
    w0t = W0.T
    w1t = W1.T
    g0r, b0r = g0.reshape(1, C0), b0.reshape(1, C0)
    g1r, b1r = g1.reshape(1, C1), b1.reshape(1, C1)
    p2_flat = points2.reshape(B * S, D2)

    grid = (B, N // BN)
    smem = pl.BlockSpec(memory_space=pltpu.SMEM)

    gidx, wn = pl.pallas_call(
        _knn_idx_body,
        grid=grid,
        in_specs=[
            smem, smem,
            pl.BlockSpec((1, BN, 3), lambda b, n: (b, n, 0)),
            pl.BlockSpec((1, 3, S), lambda b, n: (b, 0, 0)),
        ],
        out_specs=[
            pl.BlockSpec((1, BN, KP), lambda b, n: (b, n, 0)),
            pl.BlockSpec((1, BN, KP), lambda b, n: (b, n, 0)),
        ],
        out_shape=[
            jax.ShapeDtypeStruct((B, N, KP), jnp.int32),
            jax.ShapeDtypeStruct((B, N, KP), jnp.float32),
        ],
    )(point_lens, embedding_lens, xyz1, xyz2t)

    interp = _sc_gather(gidx.reshape(NW * NCHUNK, CQ * KP),
                        wn.reshape(NW * NCHUNK, CQ * KP), p2_flat)
    interp = interp.reshape(B, N, D2)

    h0, stats0 = pl.pallas_call(
        _h0_body,
        grid=grid,
        in_specs=[
            smem,
            pl.BlockSpec((1, BN, D1), lambda b, n: (b, n, 0)),
            pl.BlockSpec((1, BN, D2), lambda b, n: (b, n, 0)),
            pl.BlockSpec((D1 + D2, C0), lambda b, n: (0, 0)),
        ],
        out_specs=[
            pl.BlockSpec((1, BN, C0), lambda b, n: (b, n, 0)),
            pl.BlockSpec((8, C0), lambda b, n: (0, 0)),
        ],
        out_shape=[
            jax.ShapeDtypeStruct((B, N, C0), jnp.float32),
            jax.ShapeDtypeStruct((8, C0), jnp.float32),
        ],
    )(point_lens, points1, interp, w0t)

    h1, stats1 = pl.pallas_call(
        functools.partial(_bn_mlp_body, cin=C0, cout=C1),
        grid=grid,
        in_specs=[
            smem,
            pl.BlockSpec((1, BN, C0), lambda b, n: (b, n, 0)),
            pl.BlockSpec((8, C0), lambda b, n: (0, 0)),
            pl.BlockSpec((1, C0), lambda b, n: (0, 0)),
            pl.BlockSpec((1, C0), lambda b, n: (0, 0)),
            pl.BlockSpec((C0, C1), lambda b, n: (0, 0)),
        ],
        out_specs=[
            pl.BlockSpec((1, BN, C1), lambda b, n: (b, n, 0)),
            pl.BlockSpec((8, C1), lambda b, n: (0, 0)),
        ],
        out_shape=[
            jax.ShapeDtypeStruct((B, N, C1), jnp.float32),
            jax.ShapeDtypeStruct((8, C1), jnp.float32),
        ],
    )(point_lens, h0, stats0, g0r, b0r, w1t)

    out = pl.pallas_call(
        functools.partial(_bn_final_body, cin=C1),
        grid=grid,
        in_specs=[
            smem,
            pl.BlockSpec((1, BN, C1), lambda b, n: (b, n, 0)),
            pl.BlockSpec((8, C1), lambda b, n: (0, 0)),
            pl.BlockSpec((1, C1), lambda b, n: (0, 0)),
            pl.BlockSpec((1, C1), lambda b, n: (0, 0)),
        ],
        out_specs=pl.BlockSpec((1, BN, C1), lambda b, n: (b, n, 0)),
        out_shape=jax.ShapeDtypeStruct((B, N, C1), jnp.float32),
    )(point_lens, h1, stats1, g1r, b1r)

    return out


# speedup vs baseline: 2.1704x; 2.1669x over previous
import functools

import jax
import jax.numpy as jnp
from jax import lax
from jax.experimental import pallas as pl
from jax.experimental.pallas import tpu as pltpu
from jax.experimental.pallas import tpu_sc as plsc

B, N, S, D1, D2, K = 8, 4096, 1024, 128, 256, 5
C0, C1 = 256, 128
BN = 4096
Q = B * N
KP = 8
INF = 3e38
BIG = 1e37
EPS = 1.1920928955078125e-07

NC, NS = 2, 16
NW = NC * NS
QPW = Q // NW
CQ = 8
NCHUNK = QPW // CQ


def _knn_idx_body(plens_ref, elens_ref, xyz1_ref, xyz2t_ref, gidx_ref, wn_ref):
    b = pl.program_id(0)

    a = xyz1_ref[0]
    bt = xyz2t_ref[0]
    an = jnp.sum(a * a, axis=1, keepdims=True)
    bn_sq = jnp.sum(bt * bt, axis=0, keepdims=True)
    col_s = jax.lax.broadcasted_iota(jnp.int32, (1, S), 1)
    elen = elens_ref[b]
    bn_m = jnp.where(col_s < elen, bn_sq, BIG)
    ab2 = jax.lax.dot_general(a * (-2.0), bt, (((1,), (0,)), ((), ())),
                              precision=jax.lax.Precision.HIGHEST,
                              preferred_element_type=jnp.float32)
    d0 = jnp.maximum(ab2 + an + bn_m, 0.0)

    col = jax.lax.broadcasted_iota(jnp.int32, (BN, S), 1)
    lane8 = jax.lax.broadcasted_iota(jnp.int32, (BN, KP), 1)
    d = d0
    acc_idx = jnp.zeros((BN, KP), jnp.int32)
    acc_w = jnp.zeros((BN, KP), jnp.float32)
    for k in range(K):
        m = jnp.min(d, axis=1, keepdims=True)
        pick_col = jnp.min(jnp.where(d == m, col, S), axis=1,
                           keepdims=True)
        wk = 1.0 / (m + EPS)
        acc_idx = acc_idx + jnp.where(lane8 == k,
                                      jnp.broadcast_to(pick_col, (BN, KP)), 0)
        acc_w = acc_w + jnp.where(lane8 == k,
                                  jnp.broadcast_to(wk, (BN, KP)), 0.0)
        d = jnp.where(col == pick_col, INF, d)
    wsum = jnp.sum(acc_w, axis=1, keepdims=True)
    wn_ref[0] = acc_w / wsum
    gidx_ref[0] = acc_idx + b * S


NRING = 4


def _sc_gather_body(idx_hbm, w_hbm, table_hbm, out_hbm, idx_v, w_v, rows0_v,
                    rows1_v, rows2_v, rows3_v, out_v, sem0, sem1, sem2, sem3):
    wid = lax.axis_index("s") * NC + lax.axis_index("c")
    base_q = wid * QPW
    rows = (rows0_v, rows1_v, rows2_v, rows3_v)
    sems = (sem0, sem1, sem2, sem3)

    pltpu.sync_copy(idx_hbm.at[pl.ds(wid * NCHUNK, NCHUNK)], idx_v)
    pltpu.sync_copy(w_hbm.at[pl.ds(wid * NCHUNK, NCHUNK)], w_v)
    for c0 in range(NRING - 1):
        pltpu.async_copy(table_hbm.at[idx_v.at[c0]], rows[c0], sems[c0])

    def ring(i, carry):
        for p in range(NRING):
            c = NRING * i + p
            q0 = base_q + c * CQ
            pf = (p + NRING - 1) % NRING

            @pl.when(c + NRING - 1 < NCHUNK)
            def _():
                pltpu.async_copy(table_hbm.at[idx_v.at[c + NRING - 1]],
                                 rows[pf], sems[pf])

            pltpu.make_async_copy(table_hbm.at[idx_v.at[c]], rows[p],
                                  sems[p]).wait()

            def per_q2(q2, carry2, *, p=p, c=c):
                wv = w_v[c, pl.ds(q2 * 16, 16)]
                for half in range(2):
                    q = 2 * q2 + half
                    r0 = q * K
                    for j in range(D2 // 16):
                        sl = pl.ds(j * 16, 16)
                        h = half * KP
                        t0 = (wv[h] * rows[p][r0, sl]
                              + wv[h + 1] * rows[p][r0 + 1, sl])
                        t1 = (wv[h + 2] * rows[p][r0 + 2, sl]
                              + wv[h + 3] * rows[p][r0 + 3, sl])
                        out_v[q, sl] = ((t0 + t1)
                                        + wv[h + 4] * rows[p][r0 + 4, sl])
                return carry2

            lax.fori_loop(0, CQ // 2, per_q2, 0)
            pltpu.sync_copy(out_v, out_hbm.at[pl.ds(q0, CQ)])
        return carry

    lax.fori_loop(0, NCHUNK // NRING, ring, 0)


_sc_gather = functools.partial(
    pl.kernel,
    out_type=jax.ShapeDtypeStruct((Q, D2), jnp.float32),
    mesh=plsc.VectorSubcoreMesh(core_axis_name="c", subcore_axis_name="s"),
    scratch_types=[
        pltpu.VMEM((NCHUNK, CQ * K), jnp.int32),
        pltpu.VMEM((NCHUNK, CQ * KP), jnp.float32),
        pltpu.VMEM((CQ * K, D2), jnp.float32),
        pltpu.VMEM((CQ * K, D2), jnp.float32),
        pltpu.VMEM((CQ * K, D2), jnp.float32),
        pltpu.VMEM((CQ * K, D2), jnp.float32),
        pltpu.VMEM((CQ, D2), jnp.float32),
        pltpu.SemaphoreType.DMA,
        pltpu.SemaphoreType.DMA,
        pltpu.SemaphoreType.DMA,
        pltpu.SemaphoreType.DMA,
    ],
)(_sc_gather_body)


def _h0_body(plens_ref, p1_ref, interp_ref, w0t_ref, h0_ref, stats_ref):
    b = pl.program_id(0)
    nb = pl.program_id(1)
    w0t = w0t_ref[...]
    h0 = (jax.lax.dot_general(p1_ref[0], w0t[:D1], (((1,), (0,)), ((), ())),
                              preferred_element_type=jnp.float32)
          + jax.lax.dot_general(interp_ref[0], w0t[D1:],
                                (((1,), (0,)), ((), ())),
                                preferred_element_type=jnp.float32))
    h0_ref[0] = h0

    row = jax.lax.broadcasted_iota(jnp.int32, (BN, 1), 0) + nb * BN
    m_row = (row < plens_ref[b]).astype(jnp.float32)
    s1 = jnp.sum(h0 * m_row, axis=0, keepdims=True)
    s2 = jnp.sum(h0 * h0 * m_row, axis=0, keepdims=True)
    riota = jax.lax.broadcasted_iota(jnp.int32, (8, C0), 0)
    contrib = (jnp.where(riota == 0, jnp.broadcast_to(s1, (8, C0)), 0.0)
               + jnp.where(riota == 1, jnp.broadcast_to(s2, (8, C0)), 0.0))

    @pl.when(jnp.logical_and(b == 0, nb == 0))
    def _():
        stats_ref[...] = jnp.zeros((8, C0), jnp.float32)

    stats_ref[...] += contrib


def _n_valid(plens_ref):
    nv = jnp.int32(0)
    for i in range(B):
        nv = nv + plens_ref[i]
    return jnp.maximum(nv.astype(jnp.float32), 1.0)


def _bn_mlp_body(plens_ref, h0_ref, stats_ref, g_ref, bias_ref, w1t_ref,
                 h1_ref, stats2_ref, *, cin, cout):
    b = pl.program_id(0)
    nb = pl.program_id(1)
    nv = _n_valid(plens_ref)
    stats = stats_ref[...]
    mean = stats[0:1, :] / nv
    var = stats[1:2, :] / nv - mean * mean
    scale = g_ref[...] * jax.lax.rsqrt(var + 1e-5)
    shift = bias_ref[...] - mean * scale
    xn = jnp.maximum(h0_ref[0] * scale + shift, 0.0)
    h1 = jax.lax.dot_general(xn, w1t_ref[...], (((1,), (0,)), ((), ())),
                             preferred_element_type=jnp.float32)
    h1_ref[0] = h1

    row = jax.lax.broadcasted_iota(jnp.int32, (BN, 1), 0) + nb * BN
    m_row = (row < plens_ref[b]).astype(jnp.float32)
    s1 = jnp.sum(h1 * m_row, axis=0, keepdims=True)
    s2 = jnp.sum(h1 * h1 * m_row, axis=0, keepdims=True)
    riota = jax.lax.broadcasted_iota(jnp.int32, (8, cout), 0)
    contrib = (jnp.where(riota == 0, jnp.broadcast_to(s1, (8, cout)), 0.0)
               + jnp.where(riota == 1, jnp.broadcast_to(s2, (8, cout)), 0.0))

    @pl.when(jnp.logical_and(b == 0, nb == 0))
    def _():
        stats2_ref[...] = jnp.zeros((8, cout), jnp.float32)

    stats2_ref[...] += contrib


def _bn_final_body(plens_ref, h1_ref, stats_ref, g_ref, bias_ref, out_ref, *,
                   cin):
    nv = _n_valid(plens_ref)
    stats = stats_ref[...]
    mean = stats[0:1, :] / nv
    var = stats[1:2, :] / nv - mean * mean
    scale = g_ref[...] * jax.lax.rsqrt(var + 1e-5)
    shift = bias_ref[...] - mean * scale
    out_ref[0] = jnp.maximum(h1_ref[0] * scale + shift, 0.0)


def kernel(xyz1, xyz2, points1, points2, point_lens, embedding_lens,
           point_mask, W0, g0, b0, W1, g1, b1):
    del point_mask
    xyz2t = xyz2.transpose(0, 2, 1)
    w0t = W0.T
    w1t = W1.T
    g0r, b0r = g0.reshape(1, C0), b0.reshape(1, C0)
    g1r, b1r = g1.reshape(1, C1), b1.reshape(1, C1)
    p2_flat = points2.reshape(B * S, D2)

    grid = (B, N // BN)
    smem = pl.BlockSpec(memory_space=pltpu.SMEM)

    gidx, wn = pl.pallas_call(
        _knn_idx_body,
        grid=grid,
        in_specs=[
            smem, smem,
            pl.BlockSpec((1, BN, 3), lambda b, n: (b, n, 0)),
            pl.BlockSpec((1, 3, S), lambda b, n: (b, 0, 0)),
        ],
        out_specs=[
            pl.BlockSpec((1, BN, KP), lambda b, n: (b, n, 0)),
            pl.BlockSpec((1, BN, KP), lambda b, n: (b, n, 0)),
        ],
        out_shape=[
            jax.ShapeDtypeStruct((B, N, KP), jnp.int32),
            jax.ShapeDtypeStruct((B, N, KP), jnp.float32),
        ],
    )(point_lens, embedding_lens, xyz1, xyz2t)

    interp = _sc_gather(gidx[:, :, :K].reshape(NW * NCHUNK, CQ * K),
                        wn.reshape(NW * NCHUNK, CQ * KP), p2_flat)
    interp = interp.reshape(B, N, D2)

    h0, stats0 = pl.pallas_call(
        _h0_body,
        grid=grid,
        in_specs=[
            smem,
            pl.BlockSpec((1, BN, D1), lambda b, n: (b, n, 0)),
            pl.BlockSpec((1, BN, D2), lambda b, n: (b, n, 0)),
            pl.BlockSpec((D1 + D2, C0), lambda b, n: (0, 0)),
        ],
        out_specs=[
            pl.BlockSpec((1, BN, C0), lambda b, n: (b, n, 0)),
            pl.BlockSpec((8, C0), lambda b, n: (0, 0)),
        ],
        out_shape=[
            jax.ShapeDtypeStruct((B, N, C0), jnp.float32),
            jax.ShapeDtypeStruct((8, C0), jnp.float32),
        ],
    )(point_lens, points1, interp, w0t)

    h1, stats1 = pl.pallas_call(
        functools.partial(_bn_mlp_body, cin=C0, cout=C1),
        grid=grid,
        in_specs=[
            smem,
            pl.BlockSpec((1, BN, C0), lambda b, n: (b, n, 0)),
            pl.BlockSpec((8, C0), lambda b, n: (0, 0)),
            pl.BlockSpec((1, C0), lambda b, n: (0, 0)),
            pl.BlockSpec((1, C0), lambda b, n: (0, 0)),
            pl.BlockSpec((C0, C1), lambda b, n: (0, 0)),
        ],
        out_specs=[
            pl.BlockSpec((1, BN, C1), lambda b, n: (b, n, 0)),
            pl.BlockSpec((8, C1), lambda b, n: (0, 0)),
        ],
        out_shape=[
            jax.ShapeDtypeStruct((B, N, C1), jnp.float32),
            jax.ShapeDtypeStruct((8, C1), jnp.float32),
        ],
    )(point_lens, h0, stats0, g0r, b0r, w1t)

    out = pl.pallas_call(
        functools.partial(_bn_final_body, cin=C1),
        grid=grid,
        in_specs=[
            smem,
            pl.BlockSpec((1, BN, C1), lambda b, n: (b, n, 0)),
            pl.BlockSpec((8, C1), lambda b, n: (0, 0)),
            pl.BlockSpec((1, C1), lambda b, n: (0, 0)),
            pl.BlockSpec((1, C1), lambda b, n: (0, 0)),
        ],
        out_specs=pl.BlockSpec((1, BN, C1), lambda b, n: (b, n, 0)),
        out_shape=jax.ShapeDtypeStruct((B, N, C1), jnp.float32),
    )(point_lens, h1, stats1, g1r, b1r)

    return out
